# Initial kernel scaffold; baseline (speedup 1.0000x reference)
import jax
import jax.numpy as jnp
from jax.experimental import pallas as pl


def kernel(x_s, x_f, noise, params, edge_index_s, edge_index_f, idx):
    raise NotImplementedError("write your pallas kernel here")


# speedup vs baseline: 3.7326x; 3.7326x over previous
import functools

import jax
import jax.numpy as jnp
from jax import lax
from jax.experimental import pallas as pl
from jax.experimental.pallas import tpu as pltpu
from jax.experimental.pallas import tpu_sc as plsc

N = 10000
NP = 10240
E = 160000
EPT = 5120
NB = EPT // 128
RPT = NP // 16
PAD_NODE = 10000

_SC_MESH = plsc.VectorSubcoreMesh(core_axis_name="c", subcore_axis_name="s")


@functools.partial(
    pl.kernel, mesh=_SC_MESH,
    out_type=jax.ShapeDtypeStruct((2, 2, NP, 128), jnp.float32),
    scratch_types=[
        pltpu.VMEM((NB, 128), jnp.int32),
        pltpu.VMEM((128, 128), jnp.float32),
        pltpu.VMEM_SHARED((NP, 128), jnp.float32),
    ],
)
def _sc_degree(dst_s, dst_f, zeros128, ones128, out, dstv, onesv, accum):
    c = lax.axis_index("c")
    s = lax.axis_index("s")
    wid = s * 2 + c
    pltpu.sync_copy(ones128, onesv)
    for g, dst3 in enumerate((dst_s, dst_f)):
        @pl.when(s == 0)
        def _():
            pltpu.sync_copy(zeros128, accum)
        plsc.subcore_barrier()
        pltpu.sync_copy(dst3.at[wid], dstv)

        def body(j, carry):
            pltpu.sync_copy(onesv, accum.at[dstv.at[j]], add=True)
            return carry
        lax.fori_loop(0, NB, body, 0)
        plsc.subcore_barrier()
        pltpu.sync_copy(accum.at[pl.ds(s * RPT, RPT)],
                        out.at[g, c, pl.ds(s * RPT, RPT)])
        plsc.subcore_barrier()


def _make_sc_prop(num_chunks):
    @functools.partial(
        pl.kernel, mesh=_SC_MESH,
        out_type=(jax.ShapeDtypeStruct((2, num_chunks, NP, 128), jnp.float32),
                  jax.ShapeDtypeStruct((2, num_chunks, NP, 128), jnp.float32)),
        scratch_types=[
            pltpu.VMEM((NB, 128), jnp.int32),
            pltpu.VMEM((NB, 128), jnp.int32),
            pltpu.VMEM((2, 128, 128), jnp.float32),
            pltpu.VMEM_SHARED((NP, 128), jnp.float32),
            pltpu.SemaphoreType.DMA,
            pltpu.SemaphoreType.DMA,
        ],
    )
    def prop(y_s, y_f, src_s, dst_s, src_f, dst_f, zeros128, out_s, out_f,
             srcv, dstv, rows, accum, s0, s1):
        sems = (s0, s1)
        c = lax.axis_index("c")
        s = lax.axis_index("s")
        wid = s * 2 + c
        for y, src3, dst3, out in ((y_s, src_s, dst_s, out_s),
                                   (y_f, src_f, dst_f, out_f)):
            pltpu.sync_copy(src3.at[wid], srcv)
            pltpu.sync_copy(dst3.at[wid], dstv)
            for ch in range(num_chunks):
                @pl.when(jnp.logical_and(s == 0, c == 0))
                def _():
                    pltpu.sync_copy(y.at[ch], accum)

                @pl.when(jnp.logical_and(s == 0, c == 1))
                def _():
                    pltpu.sync_copy(zeros128, accum)
                plsc.subcore_barrier()

                ytab = y.at[ch]
                for r in range(2):
                    pltpu.async_copy(ytab.at[srcv.at[r]], rows.at[r], sems[r])

                def group(g, carry):
                    for r in range(2):
                        jj = g * 2 + r
                        pltpu.make_async_copy(
                            ytab.at[srcv.at[jj]], rows.at[r], sems[r]).wait()
                        pltpu.sync_copy(rows.at[r], accum.at[dstv.at[jj]],
                                        add=True)
                        pltpu.async_copy(
                            ytab.at[srcv.at[jj + 2]], rows.at[r], sems[r])
                    return carry
                lax.fori_loop(0, NB // 2 - 1, group, 0)
                for r in range(2):
                    jj = NB - 2 + r
                    pltpu.make_async_copy(
                        ytab.at[srcv.at[jj]], rows.at[r], sems[r]).wait()
                    pltpu.sync_copy(rows.at[r], accum.at[dstv.at[jj]],
                                    add=True)
                plsc.subcore_barrier()
                pltpu.sync_copy(accum.at[pl.ds(s * RPT, RPT)],
                                out.at[c, ch, pl.ds(s * RPT, RPT)])
                plsc.subcore_barrier()
    return prop


_sc_prop6 = _make_sc_prop(6)
_sc_prop3 = _make_sc_prop(3)


@functools.partial(
    pl.kernel, mesh=_SC_MESH,
    out_type=jax.ShapeDtypeStruct((8192, 128), jnp.float32),
    scratch_types=[
        pltpu.VMEM((2, 128), jnp.int32),
        pltpu.VMEM((128, 128), jnp.float32),
        pltpu.SemaphoreType.DMA,
    ],
)
def _sc_gather(emb, idx3, out, idxv, rows, sem):
    c = lax.axis_index("c")
    s = lax.axis_index("s")
    wid = s * 2 + c
    pltpu.sync_copy(idx3.at[wid], idxv)
    for b in range(2):
        pltpu.async_copy(emb.at[idxv.at[b]], rows, sem).wait()
        pltpu.sync_copy(rows, out.at[pl.ds(wid * 256 + b * 128, 128)])


R = 2048
GRID = NP // R


def _dinv_of(dg):
    return lax.rsqrt(1.0 + dg[0, :, 0:1] + dg[1, :, 0:1])


def _lrelu(v, a):
    return jnp.where(v > 0, v, a * v)


def _dot(a, b):
    return jnp.dot(a, b, preferred_element_type=jnp.float32)


def _p0_body(xs, xf, ns, dgs, dgf, w11, w12, w13, ys_out, yf_out):
    dinv_s = _dinv_of(dgs)
    dinv_f = _dinv_of(dgf)
    xp = xf[...] + 0.1 * ns[...]
    xw11 = _dot(xs[...], w11[...])
    xwp1 = _dot(xp, w11[...])
    xw12 = _dot(xs[...], w12[...])
    xwp2 = _dot(xp, w12[...])
    xw13 = _dot(xs[...], w13[...])
    for ch in range(2):
        sl = slice(ch * 128, (ch + 1) * 128)
        ys_out[ch] = dinv_s * xw11[:, sl]
        ys_out[2 + ch] = dinv_s * xwp1[:, sl]
        ys_out[4 + ch] = dinv_s * xw13[:, sl]
        yf_out[ch] = dinv_f * xw12[:, sl]
        yf_out[2 + ch] = dinv_f * xwp2[:, sl]
        yf_out[4 + ch] = dinv_f * xw13[:, sl]


def _tc_p0(xsp, xfp, nsp, degp_s, degp_f, params):
    full = lambda shp: pl.BlockSpec(shp, lambda i: tuple(0 for _ in shp))
    spec_x = pl.BlockSpec((R, 256), lambda i: (i, 0))
    spec_dg = pl.BlockSpec((2, R, 128), lambda i: (0, i, 0))
    spec_y = pl.BlockSpec((6, R, 128), lambda i: (0, i, 0))
    return pl.pallas_call(
        _p0_body,
        grid=(GRID,),
        in_specs=[spec_x, spec_x, spec_x, spec_dg, spec_dg] +
                 [full((256, 256))] * 3,
        out_specs=[spec_y, spec_y],
        out_shape=[jax.ShapeDtypeStruct((6, NP, 128), jnp.float32)] * 2,
    )(xsp, xfp, nsp, degp_s, degp_f,
      params['enc1']['W1'], params['enc2']['W1'], params['enc3']['W1'])


def _p2_body(z1, dg, b1a, a1a, b1c, a1c, w2a, w2c, vout):
    dinv = _dinv_of(dg)
    z = [dinv * (z1[0, ch] + z1[1, ch]) for ch in range(6)]
    g1 = _lrelu(jnp.concatenate(z[0:2], axis=1) + b1a[...], a1a[...])
    g3 = _lrelu(jnp.concatenate(z[2:4], axis=1) + b1a[...], a1a[...])
    g5 = _lrelu(jnp.concatenate(z[4:6], axis=1) + b1c[...], a1c[...])
    vout[0] = dinv * _dot(g1, w2a[...])
    vout[1] = dinv * _dot(g3, w2a[...])
    vout[2] = dinv * _dot(g5, w2c[...])


def _tc_p2(z1p, degp, enc_a, enc_c):
    full = lambda shp: pl.BlockSpec(shp, lambda i: tuple(0 for _ in shp))
    return pl.pallas_call(
        _p2_body,
        grid=(GRID,),
        in_specs=[
            pl.BlockSpec((2, 6, R, 128), lambda i: (0, 0, i, 0)),
            pl.BlockSpec((2, R, 128), lambda i: (0, i, 0)),
            full((1, 256)), full((1, 256)),
            full((1, 256)), full((1, 256)),
            full((256, 128)), full((256, 128)),
        ],
        out_specs=pl.BlockSpec((3, R, 128), lambda i: (0, i, 0)),
        out_shape=jax.ShapeDtypeStruct((3, NP, 128), jnp.float32),
    )(z1p, degp,
      enc_a['b1'].reshape(1, 256), enc_a['a1'].reshape(1, 256),
      enc_c['b1'].reshape(1, 256), enc_c['a1'].reshape(1, 256),
      enc_a['W2'], enc_c['W2'])


def _p4_body(z2s, z2f, dgs, dgf,
             b2a, a2a, b2b, a2b, b2c, a2c,
             lw, lb, aw1, ab1, aw2,
             h1o, h2o, h3o, h4o, embo, hsumo):
    dinv_s = _dinv_of(dgs)
    dinv_f = _dinv_of(dgf)
    zs = z2s[0] + z2s[1]
    zf = z2f[0] + z2f[1]
    e1 = _lrelu(dinv_s * zs[0] + b2a[...], a2a[...])
    e3 = _lrelu(dinv_s * zs[1] + b2a[...], a2a[...])
    e5 = _lrelu(dinv_s * zs[2] + b2c[...], a2c[...])
    e2 = _lrelu(dinv_f * zf[0] + b2b[...], a2b[...])
    e4 = _lrelu(dinv_f * zf[1] + b2b[...], a2b[...])
    e6 = _lrelu(dinv_f * zf[2] + b2c[...], a2c[...])
    lwv, lbv = lw[...], lb[...]
    h1 = _dot(e1, lwv) + lbv
    h2 = _dot(e2, lwv) + lbv
    h3 = _dot(e3, lwv) + lbv
    h4 = _dot(e4, lwv) + lbv
    hcom = (e5 + e6) * 0.5
    w1 = _dot(jnp.tanh(_dot(h1, aw1[...]) + ab1[...]), aw2[...])
    w2 = _dot(jnp.tanh(_dot(h2, aw1[...]) + ab1[...]), aw2[...])
    w3 = _dot(jnp.tanh(_dot(hcom, aw1[...]) + ab1[...]), aw2[...])
    m = jnp.maximum(jnp.maximum(w1, w2), w3)
    x1 = jnp.exp(w1 - m)
    x2 = jnp.exp(w2 - m)
    x3 = jnp.exp(w3 - m)
    emb = (x1 * h1 + x2 * h2 + x3 * hcom) / (x1 + x2 + x3)
    h1o[...] = h1
    h2o[...] = h2
    h3o[...] = h3
    h4o[...] = h4
    embo[...] = emb
    i = pl.program_id(0)
    mask = (lax.broadcasted_iota(jnp.int32, (R, 1), 0) + i * R) < N
    hsumo[0] = jnp.stack([
        jnp.sum(jnp.where(mask, h1, 0.0), axis=0),
        jnp.sum(jnp.where(mask, h2, 0.0), axis=0)])


def _tc_p4(z2ps, z2pf, degp_s, degp_f, params):
    full = lambda shp: pl.BlockSpec(shp, lambda i: tuple(0 for _ in shp))
    spec_z = pl.BlockSpec((2, 3, R, 128), lambda i: (0, 0, i, 0))
    spec_dg = pl.BlockSpec((2, R, 128), lambda i: (0, i, 0))
    spec_h = pl.BlockSpec((R, 128), lambda i: (i, 0))
    ea, eb, ec = params['enc1'], params['enc2'], params['enc3']
    r128 = lambda v: v.reshape(1, 128)
    return pl.pallas_call(
        _p4_body,
        grid=(GRID,),
        in_specs=[spec_z, spec_z, spec_dg, spec_dg] +
                 [full((1, 128))] * 6 +
                 [full((128, 128)), full((1, 128)),
                  full((128, 128)), full((1, 128)), full((128, 1))],
        out_specs=[spec_h] * 5 + [pl.BlockSpec((1, 2, 128), lambda i: (i, 0, 0))],
        out_shape=[jax.ShapeDtypeStruct((NP, 128), jnp.float32)] * 5 +
                  [jax.ShapeDtypeStruct((GRID, 2, 128), jnp.float32)],
    )(z2ps, z2pf, degp_s, degp_f,
      r128(ea['b2']), r128(ea['a2']), r128(eb['b2']), r128(eb['a2']),
      r128(ec['b2']), r128(ec['a2']),
      params['local_W'], r128(params['local_b']),
      params['att_W1'], r128(params['att_b1']), params['att_W2'])


def _p5a_body(h1, h2, h3, h4, wb0, cs, bb, out):
    c1 = cs[0:1, :]
    c2 = cs[1:2, :]
    cols = [jnp.sum(_dot(h[...], wb0[...]) * cc, axis=1, keepdims=True)
            for h, cc in ((h1, c1), (h2, c2), (h3, c1), (h4, c2))]
    out[...] = jnp.concatenate(cols, axis=1) + bb[...]


def _tc_p5a(h1, h2, h3, h4, wb0, cs, bb):
    full = lambda shp: pl.BlockSpec(shp, lambda i: tuple(0 for _ in shp))
    spec_h = pl.BlockSpec((R, 128), lambda i: (i, 0))
    return pl.pallas_call(
        _p5a_body,
        grid=(GRID,),
        in_specs=[spec_h] * 4 + [full((128, 128)), full((2, 128)),
                                 full((1, 1))],
        out_specs=pl.BlockSpec((R, 4), lambda i: (i, 0)),
        out_shape=jax.ShapeDtypeStruct((NP, 4), jnp.float32),
    )(h1, h2, h3, h4, wb0, cs, bb)


def _p5b_body(e_all, w1, b1, w2, b2, out):
    ea = e_all[0:4096, :]
    eb = e_all[4096:8192, :]
    feature = jnp.concatenate([ea + eb, ea * eb, ea, eb], axis=1)
    log1 = jnp.maximum(_dot(feature, w1[...]) + b1[...], 0.0)
    out[...] = _dot(log1, w2[...]) + b2[...]


def _tc_p5b(e_all, params):
    return pl.pallas_call(
        _p5b_body,
        out_shape=jax.ShapeDtypeStruct((4096, 1), jnp.float32),
    )(e_all, params['dec1_W'], params['dec1_b'].reshape(1, 256),
      params['dec2_W'], params['dec2_b'].reshape(1, 1))


def _edges3(ei):
    npad = 32 * EPT - E
    src = jnp.concatenate([ei[0].astype(jnp.int32),
                           jnp.full((npad,), PAD_NODE, jnp.int32)])
    dst = jnp.concatenate([ei[1].astype(jnp.int32),
                           jnp.full((npad,), PAD_NODE, jnp.int32)])
    return src.reshape(32, NB, 128), dst.reshape(32, NB, 128)


def _pad_rows(x):
    return jnp.pad(x, ((0, NP - N), (0, 0)))


def kernel(x_s, x_f, noise, params, edge_index_s, edge_index_f, idx):
    src_s, dst_s = _edges3(edge_index_s)
    src_f, dst_f = _edges3(edge_index_f)
    zeros128 = jnp.zeros((NP, 128), jnp.float32)
    ones128 = jnp.ones((128, 128), jnp.float32)

    degp = _sc_degree(dst_s, dst_f, zeros128, ones128)
    degp_s, degp_f = degp[0], degp[1]

    ys, yf = _tc_p0(_pad_rows(x_s), _pad_rows(x_f), _pad_rows(noise),
                    degp_s, degp_f, params)
    z1s, z1f = _sc_prop6(ys, yf, src_s, dst_s, src_f, dst_f, zeros128)

    vs = _tc_p2(z1s, degp_s, params['enc1'], params['enc3'])
    vf = _tc_p2(z1f, degp_f, params['enc2'], params['enc3'])
    z2s, z2f = _sc_prop3(vs, vf, src_s, dst_s, src_f, dst_f, zeros128)

    h1, h2, h3, h4, emb, hsums = _tc_p4(z2s, z2f, degp_s, degp_f, params)

    hsum = jnp.sum(hsums, axis=0)
    gw, gb = params['global_W'], params['global_b']
    c1 = jax.nn.sigmoid(hsum[0] / N @ gw + gb)
    c2 = jax.nn.sigmoid(hsum[1] / N @ gw + gb)
    wb0 = params['disc_W'][0]
    cs = jnp.stack([c1, c2], axis=0)
    bb = params['disc_b'].reshape(1, 1)

    lg = _tc_p5a(h1, h2, h3, h4, wb0, cs, bb)
    logits = jnp.concatenate([lg[:N, 0], lg[:N, 1], lg[:N, 2], lg[:N, 3]])

    idx_cat = jnp.concatenate([idx[0].astype(jnp.int32),
                               idx[1].astype(jnp.int32) + 386])
    e_all = _sc_gather(emb, idx_cat.reshape(32, 2, 128))
    log = _tc_p5b(e_all, params)

    return (logits, log, h1[:N], h3[:N])
